# Initial kernel scaffold; baseline (speedup 1.0000x reference)
import jax
import jax.numpy as jnp
from jax.experimental import pallas as pl


def kernel(x, W0, W1, W2, W3):
    raise NotImplementedError("write your pallas kernel here")


# speedup vs baseline: 10.9197x; 10.9197x over previous
import jax
import jax.numpy as jnp
from jax import lax
from jax.experimental import pallas as pl
from jax.experimental.pallas import tpu as pltpu
from jax.experimental.pallas import tpu_sc as plsc

_D = 64
_V = 100000
_B = 4096 * 200
_NC, _NS = 2, 16
_NW = _NC * _NS
_PER_W = _B // _NW
_C = 128
_NCHUNK = _PER_W // _C


def _sum_body(a, b, c, d, o):
    o[...] = (a[...] + b[...]) + (c[...] + d[...])


def _sum_tables(w0, w1, w2, w3):
    blk = (2000, 128)
    grid = (50000 // blk[0],)
    spec = pl.BlockSpec(blk, lambda i: (i, 0))
    return pl.pallas_call(
        _sum_body,
        grid=grid,
        in_specs=[spec] * 4,
        out_specs=spec,
        out_shape=jax.ShapeDtypeStruct((50000, 128), jnp.float32),
    )(w0, w1, w2, w3)


def _gather_body(w_hbm, x_hbm, o_hbm, idx_v, rows_v, sem):
    wid = lax.axis_index("s") * _NC + lax.axis_index("c")
    base = wid * _PER_W
    pltpu.sync_copy(x_hbm.at[pl.ds(base, _PER_W)], idx_v)

    def chunk(j, carry):
        pltpu.async_copy(
            w_hbm.at[idx_v.at[pl.ds(j * _C, _C)]], rows_v, sem
        ).wait()
        pltpu.sync_copy(rows_v, o_hbm.at[pl.ds(base + j * _C, _C)])
        return carry

    lax.fori_loop(0, _NCHUNK, chunk, 0)


_mesh = plsc.VectorSubcoreMesh(
    core_axis_name="c", subcore_axis_name="s",
    num_cores=_NC, num_subcores=_NS,
)

_gather = pl.kernel(
    _gather_body,
    out_type=jax.ShapeDtypeStruct((_B, _D), jnp.float32),
    mesh=_mesh,
    scratch_types=[
        pltpu.VMEM((_PER_W,), jnp.int32),
        pltpu.VMEM((_C, _D), jnp.float32),
        pltpu.SemaphoreType.DMA,
    ],
    compiler_params=pltpu.CompilerParams(use_tc_tiling_on_sc=False),
)


def kernel(x, W0, W1, W2, W3):
    wsum = _sum_tables(
        W0.reshape(50000, 128), W1.reshape(50000, 128),
        W2.reshape(50000, 128), W3.reshape(50000, 128),
    ).reshape(_V, _D)
    out = _gather(wsum, x.reshape(_B))
    return out.reshape(x.shape + (_D,))


# speedup vs baseline: 12.6444x; 1.1579x over previous
import jax
import jax.numpy as jnp
from jax import lax
from jax.experimental import pallas as pl
from jax.experimental.pallas import tpu as pltpu
from jax.experimental.pallas import tpu_sc as plsc

_D = 64
_V = 100000
_B = 4096 * 200
_NC, _NS = 2, 16
_NW = _NC * _NS
_PER_W = _B // _NW
_C = 128
_NCHUNK = _PER_W // _C


def _sum_body(a, b, c, d, o):
    o[...] = (a[...] + b[...]) + (c[...] + d[...])


def _sum_tables(w0, w1, w2, w3):
    blk = (2000, 128)
    grid = (50000 // blk[0],)
    spec = pl.BlockSpec(blk, lambda i: (i, 0))
    return pl.pallas_call(
        _sum_body,
        grid=grid,
        in_specs=[spec] * 4,
        out_specs=spec,
        out_shape=jax.ShapeDtypeStruct((50000, 128), jnp.float32),
    )(w0, w1, w2, w3)


_NBUF = 8
_K = 4
_NGROUP = _NCHUNK // _NBUF


def _gather_body(w_hbm, x_hbm, o_hbm, idx_v, rows_v, *sems):
    gsem, ssem = sems[:_NBUF], sems[_NBUF:]
    wid = lax.axis_index("s") * _NC + lax.axis_index("c")
    base = wid * _PER_W
    pltpu.sync_copy(x_hbm.at[pl.ds(base, _PER_W)], idx_v)

    def start_gather(j, b):
        pltpu.async_copy(
            w_hbm.at[idx_v.at[pl.ds(j * _C, _C)]], rows_v.at[b], gsem[b]
        )

    def wait_gather(b):
        pltpu.make_async_copy(
            w_hbm.at[pl.ds(0, _C)], rows_v.at[b], gsem[b]
        ).wait()

    def start_store(j, b):
        pltpu.async_copy(
            rows_v.at[b], o_hbm.at[pl.ds(base + j * _C, _C)], ssem[b]
        )

    def wait_store(b):
        pltpu.make_async_copy(
            rows_v.at[b], o_hbm.at[pl.ds(base, _C)], ssem[b]
        ).wait()

    def substep(j, b, do_prefetch, do_store_wait):
        if do_prefetch:
            bn = (b + _K) % _NBUF
            if do_store_wait:
                wait_store(bn)
            start_gather(j + _K, bn)
        wait_gather(b)
        start_store(j, b)

    for b in range(_K):
        start_gather(b, b)

    for b in range(_NBUF):
        substep(b, b, True, b >= _NBUF - _K)

    def group(g, carry):
        for b in range(_NBUF):
            substep(g * _NBUF + b, b, True, True)
        return carry

    lax.fori_loop(1, _NGROUP - 1, group, 0)

    for b in range(_NBUF):
        substep((_NGROUP - 1) * _NBUF + b, b, b < _NBUF - _K, True)

    for b in range(_NBUF):
        wait_store(b)


_mesh = plsc.VectorSubcoreMesh(
    core_axis_name="c", subcore_axis_name="s",
    num_cores=_NC, num_subcores=_NS,
)

_gather = pl.kernel(
    _gather_body,
    out_type=jax.ShapeDtypeStruct((_B, _D), jnp.float32),
    mesh=_mesh,
    scratch_types=[
        pltpu.VMEM((_PER_W,), jnp.int32),
        pltpu.VMEM((_NBUF, _C, _D), jnp.float32),
        *([pltpu.SemaphoreType.DMA] * (2 * _NBUF)),
    ],
    compiler_params=pltpu.CompilerParams(use_tc_tiling_on_sc=False),
)


def kernel(x, W0, W1, W2, W3):
    wsum = _sum_tables(
        W0.reshape(50000, 128), W1.reshape(50000, 128),
        W2.reshape(50000, 128), W3.reshape(50000, 128),
    ).reshape(_V, _D)
    out = _gather(wsum, x.reshape(_B))
    return out.reshape(x.shape + (_D,))
